# baseline (speedup 1.0000x reference)
import functools
import math

import jax
import jax.numpy as jnp
from jax import lax
from jax.experimental import pallas as pl
from jax.experimental.pallas import tpu as pltpu
from jax.experimental.pallas import tpu_sc as plsc

VOCAB = 100000
DIM = 64
D2 = DIM * 2
D4 = DIM * 4
MAX_SEQ = 512
B = 1024
S = 200
N = B * S
NW = 32
G = 80
PER_W = N // NW
NG = PER_W // G
NBUF = 3
NBLK = (NG - 2) // NBUF
LANES = 16


def _pos_table():
    position = jnp.arange(MAX_SEQ, dtype=jnp.float32)[:, None]
    div_term = jnp.exp(
        jnp.arange(0, DIM, 2).astype(jnp.float32) * (-math.log(10000.0) / DIM)
    )
    pe_sin = jnp.sin(position * div_term)
    pe_cos = jnp.cos(position * div_term)
    pe_real = jnp.zeros((MAX_SEQ, DIM), jnp.float32)
    pe_real = pe_real.at[:, 0::2].set(pe_sin)
    pe_real = pe_real.at[:, 1::2].set(pe_cos)
    pe_imag = jnp.zeros((MAX_SEQ, DIM), jnp.float32)
    pe_imag = pe_imag.at[:, 0::2].set(pe_cos)
    pe_imag = pe_imag.at[:, 1::2].set(-pe_sin)
    return jnp.concatenate([pe_real, pe_imag], axis=-1)[:S]


def _sc_embed(tok3d, omega_table, pi_table, pos):
    mesh = plsc.VectorSubcoreMesh(core_axis_name="c", subcore_axis_name="s")

    @functools.partial(
        pl.kernel,
        out_type=jax.ShapeDtypeStruct((N, D4), jnp.float32),
        mesh=mesh,
        scratch_types=[
            pltpu.VMEM((NG, G), jnp.int32),
            pltpu.VMEM((S, D2), jnp.float32),
            [pltpu.VMEM((2 * G, D2), jnp.float32)] * NBUF,
            [pltpu.SemaphoreType.DMA] * NBUF,
            [pltpu.SemaphoreType.DMA] * NBUF,
        ],
    )
    def k(tok_hbm, omega_hbm, pi_hbm, pos_hbm, out_hbm,
          idx_v, pos_v, buf_v, sem_g, sem_w):
        wid = lax.axis_index("s") * 2 + lax.axis_index("c")
        base = wid * PER_W
        pltpu.sync_copy(pos_hbm, pos_v)
        pltpu.sync_copy(tok_hbm.at[wid], idx_v)

        def gathers(g, b):
            pltpu.async_copy(omega_hbm.at[idx_v.at[g]],
                             buf_v[b].at[pl.ds(0, G)], sem_g[b])
            pltpu.async_copy(pi_hbm.at[idx_v.at[g]],
                             buf_v[b].at[pl.ds(G, G)], sem_g[b])

        def wait_gathers(b):
            pltpu.make_async_copy(
                omega_hbm.at[pl.ds(0, 2 * G)], buf_v[b], sem_g[b]).wait()

        def writes(g, b):
            r0 = base + g * G
            pltpu.async_copy(buf_v[b].at[pl.ds(0, G)],
                             out_hbm.at[pl.ds(r0, G), pl.ds(0, D2)], sem_w[b])
            pltpu.async_copy(buf_v[b].at[pl.ds(G, G)],
                             out_hbm.at[pl.ds(r0, G), pl.ds(D2, D2)], sem_w[b])

        def wait_writes(b):
            pltpu.make_async_copy(
                buf_v[b].at[pl.ds(0, G)],
                out_hbm.at[pl.ds(0, G), pl.ds(0, D2)], sem_w[b]).wait()
            pltpu.make_async_copy(
                buf_v[b].at[pl.ds(G, G)],
                out_hbm.at[pl.ds(0, G), pl.ds(D2, D2)], sem_w[b]).wait()

        def compute(g, b):
            cpo = (g * G) % S

            def row_body(j, carry2):
                pj = cpo + j
                pj = jnp.where(pj >= S, pj - S, pj)
                for h in range(D2 // LANES):
                    sl = pl.ds(h * LANES, LANES)
                    p = pos_v[pj, sl]
                    buf_v[b][j, sl] = buf_v[b][j, sl] + p
                    buf_v[b][G + j, sl] = buf_v[b][G + j, sl] + p
                return carry2

            lax.fori_loop(0, G, row_body, 0)

        def step(g, b, nb, first):
            if not first:
                wait_writes(nb)
            gathers(g + 1, nb)
            wait_gathers(b)
            compute(g, b)
            writes(g, b)

        gathers(0, 0)

        def block_body(blk, carry):
            g0 = blk * NBUF
            for u in range(NBUF):
                b = u
                nb = (u + 1) % NBUF
                if u < 2:
                    @pl.when(blk > 0)
                    def _():
                        step(g0 + u, b, nb, False)

                    @pl.when(blk == 0)
                    def _():
                        step(g0 + u, b, nb, True)
                else:
                    step(g0 + u, b, nb, False)
            return carry

        lax.fori_loop(0, NBLK, block_body, 0)

        wait_writes(1)
        gathers(NG - 1, 1)
        wait_gathers(0)
        compute(NG - 2, 0)
        writes(NG - 2, 0)
        wait_gathers(1)
        compute(NG - 1, 1)
        writes(NG - 1, 1)
        wait_writes(2)
        wait_writes(0)
        wait_writes(1)

    return k(tok3d, omega_table, pi_table, pos)


def kernel(token_ids, omega_table, pi_table):
    tok3d = token_ids.reshape(NW, NG, G).astype(jnp.int32)
    pos = _pos_table()
    out = _sc_embed(tok3d, omega_table, pi_table, pos)
    return out.reshape(B, S, D4)


# speedup vs baseline: 1.4693x; 1.4693x over previous
import functools
import math

import jax
import jax.numpy as jnp
from jax import lax
from jax.experimental import pallas as pl
from jax.experimental.pallas import tpu as pltpu
from jax.experimental.pallas import tpu_sc as plsc

VOCAB = 100000
DIM = 64
D2 = DIM * 2
D4 = DIM * 4
MAX_SEQ = 512
B = 1024
S = 200
N = B * S
NW = 32
CH = 100
G = S
PER_W = N // NW
NG = PER_W // G
NCH = PER_W // CH
IBLK = 8
NBLK = NCH // IBLK
GPB = IBLK // 2
LANES = 16
NBUF = 2


def _pos_table():
    position = jnp.arange(MAX_SEQ, dtype=jnp.float32)[:, None]
    div_term = jnp.exp(
        jnp.arange(0, DIM, 2).astype(jnp.float32) * (-math.log(10000.0) / DIM)
    )
    pe_sin = jnp.sin(position * div_term)
    pe_cos = jnp.cos(position * div_term)
    pe_real = jnp.zeros((MAX_SEQ, DIM), jnp.float32)
    pe_real = pe_real.at[:, 0::2].set(pe_sin)
    pe_real = pe_real.at[:, 1::2].set(pe_cos)
    pe_imag = jnp.zeros((MAX_SEQ, DIM), jnp.float32)
    pe_imag = pe_imag.at[:, 0::2].set(pe_cos)
    pe_imag = pe_imag.at[:, 1::2].set(-pe_sin)
    return jnp.concatenate([pe_real, pe_imag], axis=-1)[:S]


def _sc_embed(tok2d, omega_table, pi_table, pos):
    mesh = plsc.VectorSubcoreMesh(core_axis_name="c", subcore_axis_name="s")

    @functools.partial(
        pl.kernel,
        out_type=jax.ShapeDtypeStruct((N, D4), jnp.float32),
        mesh=mesh,
        scratch_types=[
            pltpu.VMEM((IBLK, CH), jnp.int32),
            pltpu.VMEM((S, D2), jnp.float32),
            [pltpu.VMEM((G, D2), jnp.float32)] * NBUF,
            [pltpu.VMEM((G, D2), jnp.float32)] * NBUF,
            [pltpu.SemaphoreType.DMA] * NBUF,
            [pltpu.SemaphoreType.DMA] * NBUF,
            [pltpu.SemaphoreType.DMA] * NBUF,
        ],
    )
    def k(tok_hbm, omega_hbm, pi_hbm, pos_hbm, out_hbm,
          idx_v, pos_v, om_v, pi_v, sem_o, sem_p, sem_w):
        wid = lax.axis_index("s") * 2 + lax.axis_index("c")
        base = wid * PER_W
        pltpu.sync_copy(pos_hbm, pos_v)

        def load_idx(blk):
            pltpu.sync_copy(tok_hbm.at[pl.ds(wid * NCH + blk * IBLK, IBLK)],
                            idx_v)

        def gathers(c, b):
            pltpu.async_copy(omega_hbm.at[idx_v.at[c]],
                             om_v[b].at[pl.ds(0, CH)], sem_o[b])
            pltpu.async_copy(omega_hbm.at[idx_v.at[c + 1]],
                             om_v[b].at[pl.ds(CH, CH)], sem_o[b])
            pltpu.async_copy(pi_hbm.at[idx_v.at[c]],
                             pi_v[b].at[pl.ds(0, CH)], sem_p[b])
            pltpu.async_copy(pi_hbm.at[idx_v.at[c + 1]],
                             pi_v[b].at[pl.ds(CH, CH)], sem_p[b])

        def wait_gathers(b):
            pltpu.make_async_copy(
                omega_hbm.at[pl.ds(0, G)], om_v[b], sem_o[b]).wait()
            pltpu.make_async_copy(
                pi_hbm.at[pl.ds(0, G)], pi_v[b], sem_p[b]).wait()

        def write_half(v, g, b, col):
            pltpu.async_copy(
                v[b], out_hbm.at[pl.ds(base + g * G, G), pl.ds(col, D2)],
                sem_w[b])

        def wait_writes(b):
            pltpu.make_async_copy(
                om_v[b], out_hbm.at[pl.ds(0, G), pl.ds(0, D2)], sem_w[b]).wait()
            pltpu.make_async_copy(
                pi_v[b], out_hbm.at[pl.ds(0, G), pl.ds(D2, D2)], sem_w[b]).wait()

        def add_pos(v, b):
            def row_body(j, carry2):
                for h in range(D2 // LANES):
                    sl = pl.ds(h * LANES, LANES)
                    v[b][j, sl] = v[b][j, sl] + pos_v[j, sl]
                return carry2

            lax.fori_loop(0, G, row_body, 0)

        load_idx(0)
        gathers(0, 0)

        def block_body(blk, carry):
            for gb in range(GPB):
                g = blk * GPB + gb
                b = gb % NBUF
                nb = (gb + 1) % NBUF
                wait_gathers(b)
                add_pos(om_v, b)
                write_half(om_v, g, b, 0)
                if gb == 0:
                    @pl.when(blk > 0)
                    def _():
                        wait_writes(nb)
                else:
                    wait_writes(nb)
                if gb == GPB - 1:
                    @pl.when(blk < NBLK - 1)
                    def _():
                        load_idx(blk + 1)
                        gathers(0, nb)
                else:
                    gathers(2 * (gb + 1), nb)
                add_pos(pi_v, b)
                write_half(pi_v, g, b, D2)
            return carry

        lax.fori_loop(0, NBLK, block_body, 0)
        wait_writes((NG - 1) % NBUF)

    return k(tok2d, omega_table, pi_table, pos)


def kernel(token_ids, omega_table, pi_table):
    tok2d = token_ids.reshape(N // CH, CH).astype(jnp.int32)
    pos = _pos_table()
    out = _sc_embed(tok2d, omega_table, pi_table, pos)
    return out.reshape(B, S, D4)


# speedup vs baseline: 1.5595x; 1.0614x over previous
import functools
import math

import jax
import jax.numpy as jnp
from jax import lax
from jax.experimental import pallas as pl
from jax.experimental.pallas import tpu as pltpu
from jax.experimental.pallas import tpu_sc as plsc

VOCAB = 100000
DIM = 64
D2 = DIM * 2
D4 = DIM * 4
MAX_SEQ = 512
B = 1024
S = 200
N = B * S
NW = 32
CH = 100
G = S
PER_W = N // NW
NG = PER_W // G
NCH = PER_W // CH
IBLK = 16
NBLK = NCH // IBLK
GPB = IBLK // 2
LANES = 16
NBUF = 2
RU = 2


def _pos_table():
    position = jnp.arange(MAX_SEQ, dtype=jnp.float32)[:, None]
    div_term = jnp.exp(
        jnp.arange(0, DIM, 2).astype(jnp.float32) * (-math.log(10000.0) / DIM)
    )
    pe_sin = jnp.sin(position * div_term)
    pe_cos = jnp.cos(position * div_term)
    pe_real = jnp.zeros((MAX_SEQ, DIM), jnp.float32)
    pe_real = pe_real.at[:, 0::2].set(pe_sin)
    pe_real = pe_real.at[:, 1::2].set(pe_cos)
    pe_imag = jnp.zeros((MAX_SEQ, DIM), jnp.float32)
    pe_imag = pe_imag.at[:, 0::2].set(pe_cos)
    pe_imag = pe_imag.at[:, 1::2].set(-pe_sin)
    return jnp.concatenate([pe_real, pe_imag], axis=-1)[:S]


def _sc_embed(tok2d, omega_table, pi_table, pos):
    mesh = plsc.VectorSubcoreMesh(core_axis_name="c", subcore_axis_name="s")

    @functools.partial(
        pl.kernel,
        out_type=jax.ShapeDtypeStruct((N, D4), jnp.float32),
        mesh=mesh,
        scratch_types=[
            pltpu.VMEM((IBLK, CH), jnp.int32),
            pltpu.VMEM((S, D2), jnp.float32),
            [pltpu.VMEM((G, D2), jnp.float32)] * NBUF,
            [pltpu.VMEM((G, D2), jnp.float32)] * NBUF,
            [pltpu.SemaphoreType.DMA] * NBUF,
            [pltpu.SemaphoreType.DMA] * NBUF,
        ],
    )
    def k(tok_hbm, omega_hbm, pi_hbm, pos_hbm, out_hbm,
          idx_v, pos_v, om_v, pi_v, sem_o, sem_p):
        wid = lax.axis_index("s") * 2 + lax.axis_index("c")
        base = wid * PER_W
        pltpu.sync_copy(pos_hbm, pos_v)

        def load_idx(blk):
            pltpu.sync_copy(tok_hbm.at[pl.ds(wid * NCH + blk * IBLK, IBLK)],
                            idx_v)

        def gathers(c, b):
            pltpu.async_copy(omega_hbm.at[idx_v.at[c]],
                             om_v[b].at[pl.ds(0, CH)], sem_o[b])
            pltpu.async_copy(omega_hbm.at[idx_v.at[c + 1]],
                             om_v[b].at[pl.ds(CH, CH)], sem_o[b])
            pltpu.async_copy(pi_hbm.at[idx_v.at[c]],
                             pi_v[b].at[pl.ds(0, CH)], sem_p[b])
            pltpu.async_copy(pi_hbm.at[idx_v.at[c + 1]],
                             pi_v[b].at[pl.ds(CH, CH)], sem_p[b])

        def wait_gathers(b):
            pltpu.make_async_copy(
                omega_hbm.at[pl.ds(0, G)], om_v[b], sem_o[b]).wait()
            pltpu.make_async_copy(
                pi_hbm.at[pl.ds(0, G)], pi_v[b], sem_p[b]).wait()

        load_idx(0)
        gathers(0, 0)

        def block_body(blk, carry):
            for gb in range(GPB):
                g = blk * GPB + gb
                b = gb % NBUF
                nb = (gb + 1) % NBUF
                if gb == GPB - 1:
                    wait_gathers(b)

                    @pl.when(blk < NBLK - 1)
                    def _():
                        load_idx(blk + 1)
                        gathers(0, nb)
                else:
                    gathers(2 * (gb + 1), nb)
                    wait_gathers(b)

                def row_body(jj, carry2):
                    for r in range(RU):
                        j = jj * RU + r
                        for h in range(D2 // LANES):
                            sl = pl.ds(h * LANES, LANES)
                            p = pos_v[j, sl]
                            om_v[b][j, sl] = om_v[b][j, sl] + p
                            pi_v[b][j, sl] = pi_v[b][j, sl] + p
                    return carry2

                lax.fori_loop(0, G // RU, row_body, 0)
                r0 = base + g * G
                pltpu.sync_copy(om_v[b], out_hbm.at[pl.ds(r0, G), pl.ds(0, D2)])
                pltpu.sync_copy(pi_v[b], out_hbm.at[pl.ds(r0, G), pl.ds(D2, D2)])
            return carry

        lax.fori_loop(0, NBLK, block_body, 0)

    return k(tok2d, omega_table, pi_table, pos)


def kernel(token_ids, omega_table, pi_table):
    tok2d = token_ids.reshape(N // CH, CH).astype(jnp.int32)
    pos = _pos_table()
    out = _sc_embed(tok2d, omega_table, pi_table, pos)
    return out.reshape(B, S, D4)


# speedup vs baseline: 1.5750x; 1.0099x over previous
import functools
import math

import jax
import jax.numpy as jnp
from jax import lax
from jax.experimental import pallas as pl
from jax.experimental.pallas import tpu as pltpu
from jax.experimental.pallas import tpu_sc as plsc

VOCAB = 100000
DIM = 64
D2 = DIM * 2
D4 = DIM * 4
MAX_SEQ = 512
B = 1024
S = 200
N = B * S
NW = 32
CH = 100
G = S
PER_W = N // NW
NG = PER_W // G
NCH = PER_W // CH
IBLK = 16
NBLK = NCH // IBLK
GPB = IBLK // 2
LANES = 16
NBUF = 2
RU = 2


def _pos_table():
    position = jnp.arange(MAX_SEQ, dtype=jnp.float32)[:, None]
    div_term = jnp.exp(
        jnp.arange(0, DIM, 2).astype(jnp.float32) * (-math.log(10000.0) / DIM)
    )
    pe_sin = jnp.sin(position * div_term)
    pe_cos = jnp.cos(position * div_term)
    pe_real = jnp.zeros((MAX_SEQ, DIM), jnp.float32)
    pe_real = pe_real.at[:, 0::2].set(pe_sin)
    pe_real = pe_real.at[:, 1::2].set(pe_cos)
    pe_imag = jnp.zeros((MAX_SEQ, DIM), jnp.float32)
    pe_imag = pe_imag.at[:, 0::2].set(pe_cos)
    pe_imag = pe_imag.at[:, 1::2].set(-pe_sin)
    return jnp.concatenate([pe_real, pe_imag], axis=-1)[:S]


def _sc_embed(tok2d, omega_table, pi_table, pos):
    mesh = plsc.VectorSubcoreMesh(core_axis_name="c", subcore_axis_name="s")

    @functools.partial(
        pl.kernel,
        out_type=jax.ShapeDtypeStruct((N, D4), jnp.float32),
        mesh=mesh,
        scratch_types=[
            pltpu.VMEM((IBLK, CH), jnp.int32),
            pltpu.VMEM((S, D2), jnp.float32),
            [pltpu.VMEM((G, D2), jnp.float32)] * NBUF,
            [pltpu.VMEM((G, D2), jnp.float32)] * NBUF,
            [pltpu.SemaphoreType.DMA] * NBUF,
            [pltpu.SemaphoreType.DMA] * NBUF,
            pltpu.SemaphoreType.DMA,
        ],
    )
    def k(tok_hbm, omega_hbm, pi_hbm, pos_hbm, out_hbm,
          idx_v, pos_v, om_v, pi_v, sem_o, sem_p, sem_w):
        wid = lax.axis_index("s") * 2 + lax.axis_index("c")
        base = wid * PER_W
        pltpu.sync_copy(pos_hbm, pos_v)

        def load_idx(blk):
            pltpu.sync_copy(tok_hbm.at[pl.ds(wid * NCH + blk * IBLK, IBLK)],
                            idx_v)

        def gathers(c, b):
            pltpu.async_copy(omega_hbm.at[idx_v.at[c]],
                             om_v[b].at[pl.ds(0, CH)], sem_o[b])
            pltpu.async_copy(omega_hbm.at[idx_v.at[c + 1]],
                             om_v[b].at[pl.ds(CH, CH)], sem_o[b])
            pltpu.async_copy(pi_hbm.at[idx_v.at[c]],
                             pi_v[b].at[pl.ds(0, CH)], sem_p[b])
            pltpu.async_copy(pi_hbm.at[idx_v.at[c + 1]],
                             pi_v[b].at[pl.ds(CH, CH)], sem_p[b])

        def wait_gathers(b):
            pltpu.make_async_copy(
                omega_hbm.at[pl.ds(0, G)], om_v[b], sem_o[b]).wait()
            pltpu.make_async_copy(
                pi_hbm.at[pl.ds(0, G)], pi_v[b], sem_p[b]).wait()

        load_idx(0)
        gathers(0, 0)

        def block_body(blk, carry):
            for gb in range(GPB):
                g = blk * GPB + gb
                b = gb % NBUF
                nb = (gb + 1) % NBUF
                if gb == GPB - 1:
                    wait_gathers(b)

                    @pl.when(blk < NBLK - 1)
                    def _():
                        load_idx(blk + 1)
                        gathers(0, nb)
                else:
                    gathers(2 * (gb + 1), nb)
                    wait_gathers(b)

                def row_body(jj, carry2):
                    for r in range(RU):
                        j = jj * RU + r
                        for h in range(D2 // LANES):
                            sl = pl.ds(h * LANES, LANES)
                            p = pos_v[j, sl]
                            om_v[b][j, sl] = om_v[b][j, sl] + p
                            pi_v[b][j, sl] = pi_v[b][j, sl] + p
                    return carry2

                lax.fori_loop(0, G // RU, row_body, 0)
                r0 = base + g * G
                cp_a = pltpu.async_copy(
                    om_v[b], out_hbm.at[pl.ds(r0, G), pl.ds(0, D2)], sem_w)
                cp_b = pltpu.async_copy(
                    pi_v[b], out_hbm.at[pl.ds(r0, G), pl.ds(D2, D2)], sem_w)
                cp_a.wait()
                cp_b.wait()
            return carry

        lax.fori_loop(0, NBLK, block_body, 0)

    return k(tok2d, omega_table, pi_table, pos)


def kernel(token_ids, omega_table, pi_table):
    tok2d = token_ids.reshape(N // CH, CH).astype(jnp.int32)
    pos = _pos_table()
    out = _sc_embed(tok2d, omega_table, pi_table, pos)
    return out.reshape(B, S, D4)


# speedup vs baseline: 1.6732x; 1.0624x over previous
import functools
import math

import jax
import jax.numpy as jnp
from jax import lax
from jax.experimental import pallas as pl
from jax.experimental.pallas import tpu as pltpu
from jax.experimental.pallas import tpu_sc as plsc

VOCAB = 100000
DIM = 64
D2 = DIM * 2
D4 = DIM * 4
MAX_SEQ = 512
B = 1024
S = 200
N = B * S
NW = 32
CH = 100
G = S
PER_W = N // NW
NG = PER_W // G
NCH = PER_W // CH
IBLK = 16
NBLK = NCH // IBLK
GPB = IBLK // 2
LANES = 16
NBUF = 2
RU = 2


def _pos_table():
    position = jnp.arange(MAX_SEQ, dtype=jnp.float32)[:, None]
    div_term = jnp.exp(
        jnp.arange(0, DIM, 2).astype(jnp.float32) * (-math.log(10000.0) / DIM)
    )
    pe_sin = jnp.sin(position * div_term)
    pe_cos = jnp.cos(position * div_term)
    pe_real = jnp.zeros((MAX_SEQ, DIM), jnp.float32)
    pe_real = pe_real.at[:, 0::2].set(pe_sin)
    pe_real = pe_real.at[:, 1::2].set(pe_cos)
    pe_imag = jnp.zeros((MAX_SEQ, DIM), jnp.float32)
    pe_imag = pe_imag.at[:, 0::2].set(pe_cos)
    pe_imag = pe_imag.at[:, 1::2].set(-pe_sin)
    return jnp.concatenate([pe_real, pe_imag], axis=-1)[:S]


def _sc_embed(tok2d, omega_table, pi_table, pos):
    mesh = plsc.VectorSubcoreMesh(core_axis_name="c", subcore_axis_name="s")

    @functools.partial(
        pl.kernel,
        out_type=jax.ShapeDtypeStruct((N, D4), jnp.float32),
        mesh=mesh,
        scratch_types=[
            pltpu.VMEM((IBLK, CH), jnp.int32),
            pltpu.VMEM((S, D2), jnp.float32),
            [pltpu.VMEM((G, D2), jnp.float32)] * NBUF,
            [pltpu.VMEM((G, D2), jnp.float32)] * NBUF,
            [pltpu.SemaphoreType.DMA] * NBUF,
            [pltpu.SemaphoreType.DMA] * NBUF,
            pltpu.SemaphoreType.DMA,
        ],
    )
    def k(tok_hbm, omega_hbm, pi_hbm, pos_hbm, out_hbm,
          idx_v, pos_v, om_v, pi_v, sem_o, sem_p, sem_w):
        wid = lax.axis_index("s") * 2 + lax.axis_index("c")
        base = wid * PER_W
        pltpu.sync_copy(pos_hbm, pos_v)

        def load_idx(blk):
            pltpu.sync_copy(tok_hbm.at[pl.ds(wid * NCH + blk * IBLK, IBLK)],
                            idx_v)

        def gathers(c, b):
            pltpu.async_copy(omega_hbm.at[idx_v.at[c]],
                             om_v[b].at[pl.ds(0, CH)], sem_o[b])
            pltpu.async_copy(omega_hbm.at[idx_v.at[c + 1]],
                             om_v[b].at[pl.ds(CH, CH)], sem_o[b])
            pltpu.async_copy(pi_hbm.at[idx_v.at[c]],
                             pi_v[b].at[pl.ds(0, CH)], sem_p[b])
            pltpu.async_copy(pi_hbm.at[idx_v.at[c + 1]],
                             pi_v[b].at[pl.ds(CH, CH)], sem_p[b])

        def wait_gathers(b):
            pltpu.make_async_copy(
                omega_hbm.at[pl.ds(0, G)], om_v[b], sem_o[b]).wait()
            pltpu.make_async_copy(
                pi_hbm.at[pl.ds(0, G)], pi_v[b], sem_p[b]).wait()

        load_idx(0)
        gathers(0, 0)

        def block_body(blk, carry):
            for gb in range(GPB):
                g = blk * GPB + gb
                b = gb % NBUF
                nb = (gb + 1) % NBUF
                if gb == GPB - 1:
                    wait_gathers(b)

                    @pl.when(blk < NBLK - 1)
                    def _():
                        load_idx(blk + 1)
                        gathers(0, nb)
                else:
                    gathers(2 * (gb + 1), nb)
                    wait_gathers(b)

                @functools.partial(plsc.parallel_loop, 0, G, unroll=RU)
                def row_body(j):
                    for h in range(D2 // LANES):
                        sl = pl.ds(h * LANES, LANES)
                        p = pos_v[j, sl]
                        om_v[b][j, sl] = om_v[b][j, sl] + p
                        pi_v[b][j, sl] = pi_v[b][j, sl] + p
                r0 = base + g * G
                cp_a = pltpu.async_copy(
                    om_v[b], out_hbm.at[pl.ds(r0, G), pl.ds(0, D2)], sem_w)
                cp_b = pltpu.async_copy(
                    pi_v[b], out_hbm.at[pl.ds(r0, G), pl.ds(D2, D2)], sem_w)
                cp_a.wait()
                cp_b.wait()
            return carry

        lax.fori_loop(0, NBLK, block_body, 0)

    return k(tok2d, omega_table, pi_table, pos)


def kernel(token_ids, omega_table, pi_table):
    tok2d = token_ids.reshape(N // CH, CH).astype(jnp.int32)
    pos = _pos_table()
    out = _sc_embed(tok2d, omega_table, pi_table, pos)
    return out.reshape(B, S, D4)
